# baseline (speedup 1.0000x reference)
import functools

import jax
import jax.numpy as jnp
from jax import lax
from jax.experimental import pallas as pl
from jax.experimental.pallas import tpu as pltpu
from jax.experimental.pallas import tpu_sc as plsc

B, S, E = 4, 2048, 768
NC, NS = 2, 16
NW = NC * NS
S_PER_W = S // NW
CH = 8
LANES = 16


def _sc_body(x_hbm, tab_hbm, out_hbm, t_v, x_v):
    wid = lax.axis_index("s") * NC + lax.axis_index("c")
    s0 = wid * S_PER_W

    def chunk_body(c, _):
        sc = s0 + c * CH
        pltpu.sync_copy(tab_hbm.at[pl.ds(sc, CH)], t_v)

        def batch_body(b, _):
            pltpu.sync_copy(x_hbm.at[b, pl.ds(sc, CH)], x_v)
            for r in range(CH):
                for i in range(E // LANES):
                    off = i * LANES
                    x_v[r, pl.ds(off, LANES)] = (
                        x_v[r, pl.ds(off, LANES)] + t_v[r, pl.ds(off, LANES)]
                    )
            pltpu.sync_copy(x_v, out_hbm.at[b, pl.ds(sc, CH)])
            return 0

        lax.fori_loop(0, B, batch_body, 0)
        return 0

    lax.fori_loop(0, S_PER_W // CH, chunk_body, 0)


_sc_call = functools.partial(
    pl.kernel,
    out_type=jax.ShapeDtypeStruct((B, S, E), jnp.float32),
    mesh=plsc.VectorSubcoreMesh(core_axis_name="c", subcore_axis_name="s"),
    scratch_types=[
        pltpu.VMEM((CH, E), jnp.float32),
        pltpu.VMEM((CH, E), jnp.float32),
    ],
)(_sc_body)


def kernel(x, pos_table):
    return _sc_call(x, pos_table)


# speedup vs baseline: 1.1064x; 1.1064x over previous
import functools

import jax
import jax.numpy as jnp
from jax import lax
from jax.experimental import pallas as pl
from jax.experimental.pallas import tpu as pltpu
from jax.experimental.pallas import tpu_sc as plsc

B, S, E = 4, 2048, 768
NC, NS = 2, 16
NW = NC * NS
S_PER_W = S // NW
CH = 8
N_CHUNKS = S_PER_W // CH
NBUF = 2
LANES = 16


def _sc_body(x_hbm, tab_hbm, out_hbm, t_v, x_v, in_sem, out_sem):
    wid = lax.axis_index("s") * NC + lax.axis_index("c")
    s0 = wid * S_PER_W

    def in_copies(c, slot):
        sc = s0 + c * CH
        cps = [pltpu.make_async_copy(
            tab_hbm.at[pl.ds(sc, CH)], t_v.at[slot], in_sem.at[slot])]
        for b in range(B):
            cps.append(pltpu.make_async_copy(
                x_hbm.at[b, pl.ds(sc, CH)], x_v.at[slot, b], in_sem.at[slot]))
        return cps

    def out_copies(c, slot):
        sc = s0 + c * CH
        return [pltpu.make_async_copy(
            x_v.at[slot, b], out_hbm.at[b, pl.ds(sc, CH)], out_sem.at[slot])
            for b in range(B)]

    for cp in in_copies(0, 0):
        cp.start()

    def chunk_body(c, _):
        slot = lax.rem(c, NBUF)
        nslot = lax.rem(c + 1, NBUF)

        @pl.when(c + 1 < N_CHUNKS)
        def _prefetch():
            @pl.when(c >= 1)
            def _drain_prev_out():
                for cp in out_copies(c - 1, nslot):
                    cp.wait()
            for cp in in_copies(c + 1, nslot):
                cp.start()

        for cp in in_copies(c, slot):
            cp.wait()

        for r in range(CH):
            for i in range(E // LANES):
                off = pl.ds(i * LANES, LANES)
                t = t_v[slot, r, off]
                for b in range(B):
                    x_v[slot, b, r, off] = x_v[slot, b, r, off] + t

        for cp in out_copies(c, slot):
            cp.start()
        return 0

    lax.fori_loop(0, N_CHUNKS, chunk_body, 0)

    for cp in out_copies(N_CHUNKS - 2, (N_CHUNKS - 2) % NBUF):
        cp.wait()
    for cp in out_copies(N_CHUNKS - 1, (N_CHUNKS - 1) % NBUF):
        cp.wait()


_sc_call = functools.partial(
    pl.kernel,
    out_type=jax.ShapeDtypeStruct((B, S, E), jnp.float32),
    mesh=plsc.VectorSubcoreMesh(core_axis_name="c", subcore_axis_name="s"),
    scratch_types=[
        pltpu.VMEM((NBUF, CH, E), jnp.float32),
        pltpu.VMEM((NBUF, B, CH, E), jnp.float32),
        pltpu.SemaphoreType.DMA((NBUF,)),
        pltpu.SemaphoreType.DMA((NBUF,)),
    ],
)(_sc_body)


def kernel(x, pos_table):
    return _sc_call(x, pos_table)


# speedup vs baseline: 1.8736x; 1.6933x over previous
import functools

import jax
import jax.numpy as jnp
from jax import lax
from jax.experimental import pallas as pl
from jax.experimental.pallas import tpu as pltpu
from jax.experimental.pallas import tpu_sc as plsc

B, S, E = 4, 2048, 768
NC, NS = 2, 16
NW = NC * NS
S_PER_W = S // NW
CH = 8
N_CHUNKS = S_PER_W // CH
NBUF = 3
LANES = 16


def _sc_body(x_hbm, tab_hbm, out_hbm, t_v, x_v, in_sem, out_sem):
    wid = lax.axis_index("s") * NC + lax.axis_index("c")
    s0 = wid * S_PER_W

    def in_copies(c, slot):
        sc = s0 + c * CH
        return [
            pltpu.make_async_copy(
                tab_hbm.at[pl.ds(sc, CH)], t_v.at[slot], in_sem.at[slot]),
            pltpu.make_async_copy(
                x_hbm.at[:, pl.ds(sc, CH)], x_v.at[slot], in_sem.at[slot]),
        ]

    def out_copies(c, slot):
        sc = s0 + c * CH
        return [pltpu.make_async_copy(
            x_v.at[slot], out_hbm.at[:, pl.ds(sc, CH)], out_sem.at[slot])]

    for cp in in_copies(0, 0):
        cp.start()

    def chunk_body(c, _):
        slot = lax.rem(c, NBUF)

        @pl.when(c + 1 < N_CHUNKS)
        def _prefetch():
            nslot = lax.rem(c + 1, NBUF)

            @pl.when(c >= 2)
            def _drain_prev_out():
                for cp in out_copies(c - 2, nslot):
                    cp.wait()

            for cp in in_copies(c + 1, nslot):
                cp.start()

        for cp in in_copies(c, slot):
            cp.wait()

        for r in range(CH):
            for i in range(E // LANES):
                off = pl.ds(i * LANES, LANES)
                t = t_v[slot, r, off]
                for b in range(B):
                    plsc.addupdate(x_v.at[slot, b, r, off], t)

        for cp in out_copies(c, slot):
            cp.start()
        return 0

    lax.fori_loop(0, N_CHUNKS, chunk_body, 0)

    for c in (N_CHUNKS - 3, N_CHUNKS - 2, N_CHUNKS - 1):
        for cp in out_copies(c, c % NBUF):
            cp.wait()


_sc_call = functools.partial(
    pl.kernel,
    out_type=jax.ShapeDtypeStruct((B, S, E), jnp.float32),
    mesh=plsc.VectorSubcoreMesh(core_axis_name="c", subcore_axis_name="s"),
    scratch_types=[
        pltpu.VMEM((NBUF, CH, E), jnp.float32),
        pltpu.VMEM((NBUF, B, CH, E), jnp.float32),
        pltpu.SemaphoreType.DMA((NBUF,)),
        pltpu.SemaphoreType.DMA((NBUF,)),
    ],
)(_sc_body)


def kernel(x, pos_table):
    return _sc_call(x, pos_table)
